# baseline (device time: 90073 ns/iter reference)
import jax
import jax.numpy as jnp
from jax import lax
from jax.experimental import pallas as pl
from jax.experimental.pallas import tpu as pltpu

N_DEV = 8
N_FULL = 3
N_MSG = 2 * N_FULL + 1


def kernel(x, w_mat, scale_x, scale_w):
    m_per, k = x.shape
    _, n_per = w_mat.shape
    half = m_per // 2

    def body(x_ref, w_ref, sx_ref, sw_ref, out_ref,
             cw_ref, ccw_ref, far_ref,
             cw_send, cw_recv, ccw_send, ccw_recv):
        my = lax.axis_index("i")
        right = lax.rem(my + 1, N_DEV)
        left = lax.rem(my + N_DEV - 1, N_DEV)

        barrier_sem = pltpu.get_barrier_semaphore()
        pl.semaphore_signal(barrier_sem, inc=1, device_id=(left,),
                            device_id_type=pl.DeviceIdType.MESH)
        pl.semaphore_signal(barrier_sem, inc=1, device_id=(right,),
                            device_id_type=pl.DeviceIdType.MESH)
        pl.semaphore_wait(barrier_sem, 2)

        scale = sx_ref[0] * sw_ref[0]

        def make(slots, sends, recvs, nbr, pri, i):
            oth = half - pri
            if i == 0:
                src, dst = x_ref.at[pl.ds(pri, half)], slots.at[0, pl.ds(pri, half)]
            elif i == 1:
                src, dst = x_ref.at[pl.ds(oth, half)], slots.at[0, pl.ds(oth, half)]
            elif i < 2 * N_FULL:
                d, is_oth = divmod(i, 2)
                off = oth if is_oth else pri
                src = slots.at[d - 1, pl.ds(off, half)]
                dst = slots.at[d, pl.ds(off, half)]
            else:
                src = slots.at[N_FULL - 1, pl.ds(pri, half)]
                dst = far_ref.at[pl.ds(pri, half)]
            return pltpu.make_async_remote_copy(
                src_ref=src, dst_ref=dst,
                send_sem=sends.at[i], recv_sem=recvs.at[i],
                device_id=(nbr,), device_id_type=pl.DeviceIdType.MESH,
            )

        def cw(i):
            return make(cw_ref, cw_send, cw_recv, right, 0, i)

        def ccw(i):
            return make(ccw_ref, ccw_send, ccw_recv, left, half, i)

        def gemm(chunk, origin):
            acc = lax.dot_general(
                chunk, w_ref[...],
                (((1,), (0,)), ((), ())),
                preferred_element_type=jnp.int32,
            )
            out_ref[pl.ds(origin * m_per, m_per), :] = acc.astype(jnp.float32) * scale

        cw(0).start()
        ccw(0).start()
        cw(1).start()
        ccw(1).start()


        for d in range(N_FULL):
            r0, r1 = 2 * d, 2 * d + 1
            cw(r0).wait_recv()
            cw(r0 + 2).start()
            ccw(r0).wait_recv()
            ccw(r0 + 2).start()
            cw(r1).wait_recv()
            if r1 + 2 < N_MSG:
                cw(r1 + 2).start()
            ccw(r1).wait_recv()
            if r1 + 2 < N_MSG:
                ccw(r1 + 2).start()

        cw(N_MSG - 1).wait_recv()
        ccw(N_MSG - 1).wait_recv()
        out_ref[...] = jnp.zeros_like(out_ref)

        for i in range(N_MSG):
            cw(i).wait_send()
            ccw(i).wait_send()

    out_shape = jax.ShapeDtypeStruct((N_DEV * m_per, n_per), jnp.float32)
    return pl.pallas_call(
        body,
        out_shape=out_shape,
        in_specs=[
            pl.BlockSpec(memory_space=pltpu.VMEM),
            pl.BlockSpec(memory_space=pltpu.VMEM),
            pl.BlockSpec(memory_space=pltpu.SMEM),
            pl.BlockSpec(memory_space=pltpu.SMEM),
        ],
        out_specs=pl.BlockSpec(memory_space=pltpu.VMEM),
        scratch_shapes=[
            pltpu.VMEM((N_FULL, m_per, k), jnp.int8),
            pltpu.VMEM((N_FULL, m_per, k), jnp.int8),
            pltpu.VMEM((m_per, k), jnp.int8),
            pltpu.SemaphoreType.DMA((N_MSG,)),
            pltpu.SemaphoreType.DMA((N_MSG,)),
            pltpu.SemaphoreType.DMA((N_MSG,)),
            pltpu.SemaphoreType.DMA((N_MSG,)),
        ],
        compiler_params=pltpu.CompilerParams(collective_id=0),
    )(x, w_mat, scale_x, scale_w)


# device time: 11461 ns/iter; 7.8591x vs baseline; 7.8591x over previous
import jax
import jax.numpy as jnp
from jax import lax
from jax.experimental import pallas as pl
from jax.experimental.pallas import tpu as pltpu

N_DEV = 8
N_MSG = 7

A_REL = [(1, 0, 0), (0, 1, 0), (1, 1, 0), (0, 0, 1), (1, 0, 1), (0, 1, 1), (1, 1, 1)]
B_REL = [(0, 0, 1), (0, 1, 0), (0, 1, 1), (1, 0, 0), (1, 0, 1), (1, 1, 0), (1, 1, 1)]

A_SEND = [(None, 'x'), (None, 'y'), (0, 'y'), (None, 'z'), (0, 'z'), (1, 'z'), (2, 'z')]
B_SEND = [(None, 'z'), (None, 'y'), (0, 'y'), (None, 'x'), (0, 'x'), (1, 'x'), (2, 'x')]


def kernel(x, w_mat, scale_x, scale_w):
    m_per, k = x.shape
    _, n_per = w_mat.shape
    half = m_per // 2

    def body(x_ref, w_ref, sx_ref, sw_ref, out_ref,
             buf_a, buf_b, send_a, recv_a, send_b, recv_b):
        my = lax.axis_index("i")

        mz = my // 4
        m4 = lax.rem(my, 4)
        mx = lax.rem((m4 + 1) // 2, 2)
        yy = m4 // 2

        def pos(px, py, pz):
            return 4 * pz + px + py * (3 - 2 * px)

        nbr = {
            'x': pos(1 - mx, yy, mz),
            'y': pos(mx, 1 - yy, mz),
            'z': pos(mx, yy, 1 - mz),
        }

        def origin_pos(rel):
            dx, dy, dz = rel
            ox = (1 - mx) if dx else mx
            oy = (1 - yy) if dy else yy
            oz = (1 - mz) if dz else mz
            return pos(ox, oy, oz)

        barrier_sem = pltpu.get_barrier_semaphore()
        for d in ('x', 'y', 'z'):
            pl.semaphore_signal(barrier_sem, inc=1, device_id=(nbr[d],),
                                device_id_type=pl.DeviceIdType.MESH)
        pl.semaphore_wait(barrier_sem, 3)

        scale = sx_ref[0] * sw_ref[0]

        def make(sub, i):
            buf, sends, recvs, table, off = (
                (buf_a, send_a, recv_a, A_SEND, 0) if sub == 'A'
                else (buf_b, send_b, recv_b, B_SEND, half))
            src_slot, dim = table[i]
            if src_slot is None:
                src = x_ref.at[pl.ds(off, half)]
            else:
                src = buf.at[src_slot]
            return pltpu.make_async_remote_copy(
                src_ref=src, dst_ref=buf.at[i],
                send_sem=sends.at[i], recv_sem=recvs.at[i],
                device_id=(nbr[dim],), device_id_type=pl.DeviceIdType.MESH,
            )

        def gemm_rows(chunk, row0):
            acc = lax.dot_general(
                chunk, w_ref[...],
                (((1,), (0,)), ((), ())),
                preferred_element_type=jnp.int32,
            )
            out_ref[pl.ds(row0, chunk.shape[0]), :] = acc.astype(jnp.float32) * scale

        def gemm(sub, i):
            buf, rel, off = ((buf_a, A_REL[i], 0) if sub == 'A'
                             else (buf_b, B_REL[i], half))
            gemm_rows(buf[i], origin_pos(rel) * m_per + off)

        for sub, i in [('A', 0), ('B', 0), ('A', 1), ('B', 1), ('B', 3), ('A', 3)]:
            make(sub, i).start()
        gemm_rows(x_ref[...], my * m_per)

        make('A', 0).wait_recv()
        make('A', 2).start()
        make('A', 4).start()
        make('B', 0).wait_recv()
        make('B', 2).start()
        make('B', 4).start()
        make('A', 1).wait_recv()
        make('A', 5).start()
        for g in [('A', 0), ('B', 0), ('A', 1)]:
            gemm(*g)

        make('B', 1).wait_recv()
        make('B', 5).start()
        make('B', 3).wait_recv()
        make('A', 3).wait_recv()
        for g in [('B', 1), ('B', 3), ('A', 3)]:
            gemm(*g)

        make('A', 2).wait_recv()
        make('A', 6).start()
        make('B', 4).wait_recv()
        make('A', 4).wait_recv()
        for g in [('A', 2), ('B', 4), ('A', 4)]:
            gemm(*g)

        make('B', 2).wait_recv()
        make('B', 6).start()
        make('B', 5).wait_recv()
        make('A', 5).wait_recv()
        for g in [('B', 2), ('B', 5), ('A', 5)]:
            gemm(*g)

        make('A', 6).wait_recv()
        make('B', 6).wait_recv()
        gemm('A', 6)
        gemm('B', 6)

        for i in range(N_MSG):
            make('A', i).wait_send()
            make('B', i).wait_send()

    out_shape = jax.ShapeDtypeStruct((N_DEV * m_per, n_per), jnp.float32)
    return pl.pallas_call(
        body,
        out_shape=out_shape,
        in_specs=[
            pl.BlockSpec(memory_space=pltpu.VMEM),
            pl.BlockSpec(memory_space=pltpu.VMEM),
            pl.BlockSpec(memory_space=pltpu.SMEM),
            pl.BlockSpec(memory_space=pltpu.SMEM),
        ],
        out_specs=pl.BlockSpec(memory_space=pltpu.VMEM),
        scratch_shapes=[
            pltpu.VMEM((N_MSG, half, k), jnp.int8),
            pltpu.VMEM((N_MSG, half, k), jnp.int8),
            pltpu.SemaphoreType.DMA((N_MSG,)),
            pltpu.SemaphoreType.DMA((N_MSG,)),
            pltpu.SemaphoreType.DMA((N_MSG,)),
            pltpu.SemaphoreType.DMA((N_MSG,)),
        ],
        compiler_params=pltpu.CompilerParams(collective_id=0),
    )(x, w_mat, scale_x, scale_w)
